# baseline (speedup 1.0000x reference)
import functools

import jax
import jax.numpy as jnp
from jax import lax
from jax.experimental import pallas as pl
from jax.experimental.pallas import tpu as pltpu
from jax.experimental.pallas import tpu_sc as plsc

_NC = 2
_NS = 16
_LANES = 16
_CHUNK = 128
_G = 2
_TRASH = 128
_DEGW = 16


@functools.lru_cache(maxsize=None)
def _sc_agg_call(n_rows, n_edges, n_dst):
    nchunks = n_edges // (_NS * _CHUNK)
    assert nchunks * _NS * _CHUNK == n_edges
    g = min(_G, nchunks)
    ngroups = nchunks // g
    assert ngroups * g == nchunks and (ngroups == 1 or ngroups % 2 == 0)
    half = n_dst // _NC
    assert half * _NC == n_dst
    nacc = half + _TRASH
    zpt = nacc // _NS
    assert zpt * _NS == nacc and zpt % 8 == 0
    rpt = half // _NS
    assert rpt * _NS == half
    mesh = plsc.VectorSubcoreMesh(core_axis_name="c", subcore_axis_name="s")

    def body(x_hbm, src_hbm, dst_hbm, z_hbm,
             agg_hbm,
             sidxA, didxA, sidxB, didxB, rows, zrow,
             acc, psemA, psemB, gsem, ssem):
        c = lax.axis_index("c")
        s = lax.axis_index("s")
        row0 = s * nchunks
        cbase = c * half

        def start_prefetch(grp, sbuf, dbuf, sem):
            base = row0 + grp * g
            pltpu.async_copy(src_hbm.at[pl.ds(base, g)], sbuf, sem)
            pltpu.async_copy(dst_hbm.at[pl.ds(base, g)], dbuf, sem)

        def wait_prefetch(sbuf, dbuf, sem):
            pltpu.make_async_copy(src_hbm.at[pl.ds(0, g)], sbuf, sem).wait()
            pltpu.make_async_copy(dst_hbm.at[pl.ds(0, g)], dbuf, sem).wait()

        def transform(dbuf):
            for j in range(g):
                for k in range(_CHUNK // _LANES):
                    sl = pl.ds(k * _LANES, _LANES)
                    v = dbuf[j, sl]
                    t = v - cbase
                    oob = (t < 0) | (t >= half)
                    dbuf[j, sl] = jnp.where(
                        oob, half + (v & (_TRASH - 1)), t)

        def rows_at(j):
            return rows.at[j]

        def fire_gathers(sbuf):
            for j in range(g):
                pltpu.async_copy(x_hbm.at[sbuf.at[j]], rows_at(j), gsem)

        def drain_and_scatter(sbuf, dbuf):
            for j in range(g):
                pltpu.make_async_copy(
                    x_hbm.at[sbuf.at[j]], rows_at(j), gsem).wait()
                pltpu.async_copy(rows_at(j), acc.at[dbuf.at[j]], ssem,
                                 add=True)

        def drain_scatters(dbuf):
            for j in range(g):
                pltpu.make_async_copy(
                    rows_at(j), acc.at[dbuf.at[j]], ssem).wait()

        start_prefetch(0, sidxA, didxA, psemA)
        if ngroups > 1:
            start_prefetch(1, sidxB, didxB, psemB)

        pltpu.sync_copy(z_hbm, zrow)
        nz, tail = zpt // 64, zpt % 64

        def zbody(jz, carry):
            pltpu.sync_copy(zrow, acc.at[pl.ds(s * zpt + jz * 64, 64)])
            return carry
        if nz:
            lax.fori_loop(0, nz, zbody, 0)
        if tail:
            pltpu.sync_copy(zrow.at[pl.ds(0, tail)],
                            acc.at[pl.ds(s * zpt + nz * 64, tail)])
        plsc.subcore_barrier()

        wait_prefetch(sidxA, didxA, psemA)
        transform(didxA)

        if ngroups == 1:
            fire_gathers(sidxA)
            drain_and_scatter(sidxA, didxA)
            drain_scatters(didxA)
        else:
            def half_step(grp, sbuf, dbuf, psem, osbuf, odbuf, opsem):
                fire_gathers(sbuf)

                @pl.when(grp + 1 < ngroups)
                def _():
                    wait_prefetch(osbuf, odbuf, opsem)
                transform(odbuf)
                drain_and_scatter(sbuf, dbuf)
                drain_scatters(dbuf)

                @pl.when(grp + 2 < ngroups)
                def _():
                    start_prefetch(grp + 2, sbuf, dbuf, psem)

            def pair_body(p, carry):
                g0 = 2 * p
                half_step(g0, sidxA, didxA, psemA, sidxB, didxB, psemB)
                half_step(g0 + 1, sidxB, didxB, psemB, sidxA, didxA, psemA)
                return carry
            lax.fori_loop(0, ngroups // 2, pair_body, 0)
        plsc.subcore_barrier()

        pltpu.sync_copy(acc.at[pl.ds(s * rpt, rpt)],
                        agg_hbm.at[pl.ds(cbase + s * rpt, rpt)])

    return pl.kernel(
        body,
        out_type=jax.ShapeDtypeStruct((n_dst, 128), jnp.float32),
        mesh=mesh,
        scratch_types=[
            pltpu.VMEM((g, _CHUNK), jnp.int32),
            pltpu.VMEM((g, _CHUNK), jnp.int32),
            pltpu.VMEM((g, _CHUNK), jnp.int32),
            pltpu.VMEM((g, _CHUNK), jnp.int32),
            pltpu.VMEM((g, _CHUNK, 128), jnp.float32),
            pltpu.VMEM((64, 128), jnp.float32),
            pltpu.VMEM_SHARED((nacc, 128), jnp.float32),
            pltpu.SemaphoreType.DMA,
            pltpu.SemaphoreType.DMA,
            pltpu.SemaphoreType.DMA,
            pltpu.SemaphoreType.DMA,
        ],
    )


@functools.lru_cache(maxsize=None)
def _sc_deg_call(n_edges1, n_dst1, n_edges2, n_dst2):
    specs = []
    for n_edges, n_dst in ((n_edges1, n_dst1), (n_edges2, n_dst2)):
        rows_tile = n_edges // (_NC * _NS * _CHUNK)
        assert rows_tile * _NC * _NS * _CHUNK == n_edges
        g = min(8, rows_tile)
        ngroups = rows_tile // g
        assert ngroups * g == rows_tile
        rpt = n_dst // _NS
        assert rpt * _NS == n_dst and rpt % 8 == 0
        specs.append((rows_tile, g, ngroups, rpt, n_dst))
    mesh = plsc.VectorSubcoreMesh(core_axis_name="c", subcore_axis_name="s")

    def body(dst1_hbm, dst2_hbm, z16_hbm, o16_hbm,
             deg1_hbm, deg2_hbm,
             dbuf, onesv, z16v, dacc1, dacc2, dsem):
        c = lax.axis_index("c")
        s = lax.axis_index("s")
        pltpu.sync_copy(z16_hbm, z16v)
        pltpu.sync_copy(o16_hbm, onesv)

        for (rows_tile, g, ngroups, rpt, n_dst), dst_hbm, dacc, deg_hbm in (
                (specs[0], dst1_hbm, dacc1, deg1_hbm),
                (specs[1], dst2_hbm, dacc2, deg2_hbm)):
            nz, tail = rpt // 128, rpt % 128

            def zbody(jz, carry):
                pltpu.sync_copy(z16v,
                                dacc.at[pl.ds(s * rpt + jz * 128, 128)])
                return carry
            if nz:
                lax.fori_loop(0, nz, zbody, 0)
            if tail:
                pltpu.sync_copy(z16v.at[pl.ds(0, tail)],
                                dacc.at[pl.ds(s * rpt + nz * 128, tail)])
        plsc.subcore_barrier()

        for (rows_tile, g, ngroups, rpt, n_dst), dst_hbm, dacc, deg_hbm in (
                (specs[0], dst1_hbm, dacc1, deg1_hbm),
                (specs[1], dst2_hbm, dacc2, deg2_hbm)):
            row0 = (c * _NS + s) * rows_tile

            def gbody(i, carry):
                pltpu.sync_copy(
                    dst_hbm.at[pl.ds(row0 + i * g, g)], dbuf.at[pl.ds(0, g)])
                for j in range(g):
                    pltpu.async_copy(onesv, dacc.at[dbuf.at[j]], dsem,
                                     add=True)
                for j in range(g):
                    pltpu.make_async_copy(onesv, dacc.at[dbuf.at[0]],
                                          dsem).wait()
                return carry
            lax.fori_loop(0, ngroups, gbody, 0)
        plsc.subcore_barrier()

        for (rows_tile, g, ngroups, rpt, n_dst), dst_hbm, dacc, deg_hbm in (
                (specs[0], dst1_hbm, dacc1, deg1_hbm),
                (specs[1], dst2_hbm, dacc2, deg2_hbm)):
            pltpu.sync_copy(dacc.at[pl.ds(s * rpt, rpt)],
                            deg_hbm.at[c, pl.ds(s * rpt, rpt)])

    return pl.kernel(
        body,
        out_type=[
            jax.ShapeDtypeStruct((_NC, n_dst1, _DEGW), jnp.float32),
            jax.ShapeDtypeStruct((_NC, n_dst2, _DEGW), jnp.float32),
        ],
        mesh=mesh,
        scratch_types=[
            pltpu.VMEM((8, _CHUNK), jnp.int32),
            pltpu.VMEM((_CHUNK, _DEGW), jnp.float32),
            pltpu.VMEM((128, _DEGW), jnp.float32),
            pltpu.VMEM_SHARED((n_dst1, _DEGW), jnp.float32),
            pltpu.VMEM_SHARED((n_dst2, _DEGW), jnp.float32),
            pltpu.SemaphoreType.DMA,
        ],
        compiler_params=pltpu.CompilerParams(use_tc_tiling_on_sc=False),
    )


@functools.lru_cache(maxsize=None)
def _dense_call(n_rows, relu, blk):
    def body(xd, ag, dg_a, dg_b, ws, wn, b, out):
        m = jnp.dot(ag[...], wn[...], preferred_element_type=jnp.float32)
        deg = jnp.maximum(dg_a[...][:, 0:1] + dg_b[...][:, 0:1], 1.0)
        r = (jnp.dot(xd[...], ws[...], preferred_element_type=jnp.float32)
             + m / deg + b[...])
        out[...] = jnp.maximum(r, 0.0) if relu else r

    return pl.pallas_call(
        body,
        grid=(n_rows // blk,),
        in_specs=[
            pl.BlockSpec((blk, 128), lambda i: (i, 0)),
            pl.BlockSpec((blk, 128), lambda i: (i, 0)),
            pl.BlockSpec((blk, _DEGW), lambda i: (i, 0)),
            pl.BlockSpec((blk, _DEGW), lambda i: (i, 0)),
            pl.BlockSpec((128, 128), lambda i: (0, 0)),
            pl.BlockSpec((128, 128), lambda i: (0, 0)),
            pl.BlockSpec((1, 128), lambda i: (0, 0)),
        ],
        out_specs=pl.BlockSpec((blk, 128), lambda i: (i, 0)),
        out_shape=jax.ShapeDtypeStruct((n_rows, 128), jnp.float32),
    )


def kernel(x, src0, dst0, src1, dst1, n_dst0, n_dst1,
           W_self1, W_neigh1, b1, W_self2, W_neigh2, b2):
    del n_dst0, n_dst1
    n1 = src1.shape[0]
    n2 = 1024
    f32 = jnp.float32
    z128 = jnp.zeros((64, 128), f32)
    z16 = jnp.zeros((128, _DEGW), f32)
    o16 = jnp.ones((_CHUNK, _DEGW), f32)
    src0i = src0.astype(jnp.int32).reshape(-1, _CHUNK)
    dst0i = dst0.astype(jnp.int32).reshape(-1, _CHUNK)
    src1i = src1.astype(jnp.int32).reshape(-1, _CHUNK)
    dst1i = dst1.astype(jnp.int32).reshape(-1, _CHUNK)

    deg1, deg2 = _sc_deg_call(src0.shape[0], n1, src1.shape[0], n2)(
        dst0i, dst1i, z16, o16)
    agg1 = _sc_agg_call(x.shape[0], src0.shape[0], n1)(
        x, src0i, dst0i, z128)
    h1 = _dense_call(n1, True, 2048)(
        x[:n1], agg1, deg1[0], deg1[1],
        W_self1, W_neigh1, b1.reshape(1, 128))
    agg2 = _sc_agg_call(n1, src1.shape[0], n2)(
        h1, src1i, dst1i, z128)
    out = _dense_call(n2, False, 1024)(
        h1[:n2], agg2, deg2[0], deg2[1],
        W_self2, W_neigh2, b2.reshape(1, 128))
    return out


# speedup vs baseline: 1.3221x; 1.3221x over previous
import functools

import jax
import jax.numpy as jnp
from jax import lax
from jax.experimental import pallas as pl
from jax.experimental.pallas import tpu as pltpu
from jax.experimental.pallas import tpu_sc as plsc

_NC = 2
_NS = 16
_LANES = 16
_CHUNK = 128
_G = 2
_RING = 4
_DEGW = 16
_HALF = 64


@functools.lru_cache(maxsize=None)
def _sc_agg_call(n_half_rows, n_edges, n_dst):
    nchunks = n_edges // (_NS * _CHUNK)
    assert nchunks * _NS * _CHUNK == n_edges
    g = min(_G, nchunks)
    ngroups = nchunks // g
    assert ngroups * g == nchunks and ngroups % _RING == 0
    rpt = n_dst // _NS
    assert rpt * _NS == n_dst
    zc = min(128, rpt)
    assert rpt % zc == 0
    mesh = plsc.VectorSubcoreMesh(core_axis_name="c", subcore_axis_name="s")

    def body(x_hbm, src_hbm, dst_hbm, z64_hbm, z16_hbm, o16_hbm,
             agg_hbm, deg_hbm,
             sidx, didx, rowsE, rowsO, onesv, zrow, z16v,
             acc, dacc, psem, gsem, ssem, dsem):
        c = lax.axis_index("c")
        s = lax.axis_index("s")
        row0 = s * nchunks
        rowbufs = (rowsE, rowsO)

        def start_prefetch(grp, b):
            base = row0 + grp * g
            pltpu.async_copy(src_hbm.at[pl.ds(base, g)], sidx.at[b], psem)
            pltpu.async_copy(dst_hbm.at[pl.ds(base, g)], didx.at[b], psem)

        def wait_prefetch(b):
            pltpu.make_async_copy(
                src_hbm.at[pl.ds(0, g)], sidx.at[b], psem).wait()
            pltpu.make_async_copy(
                dst_hbm.at[pl.ds(0, g)], didx.at[b], psem).wait()

        def transform(b):
            for j in range(g):
                for k in range(_CHUNK // _LANES):
                    sl = pl.ds(k * _LANES, _LANES)
                    sidx[b, j, sl] = sidx[b, j, sl] * 2 + c

        def fire_gathers(b, rows):
            for j in range(g):
                pltpu.async_copy(x_hbm.at[sidx.at[b, j]], rows.at[j], gsem)

        def drain_and_scatter(b, rows):
            for j in range(g):
                pltpu.make_async_copy(
                    x_hbm.at[sidx.at[b, j]], rows.at[j], gsem).wait()
                pltpu.async_copy(rows.at[j], acc.at[didx.at[b, j]], ssem,
                                 add=True)
                if j % 2 == 0:
                    @pl.when(c == 0)
                    def _():
                        pltpu.async_copy(onesv, dacc.at[didx.at[b, j]],
                                         dsem, add=True)
                else:
                    @pl.when(c == 1)
                    def _():
                        pltpu.async_copy(onesv, dacc.at[didx.at[b, j]],
                                         dsem, add=True)

        def drain_scatters(b, rows):
            for j in range(g):
                pltpu.make_async_copy(
                    rows.at[j], acc.at[didx.at[b, j]], ssem).wait()
            pltpu.make_async_copy(onesv, dacc.at[didx.at[b, 0]],
                                  dsem).wait()

        for grp in range(min(_RING - 1, ngroups)):
            start_prefetch(grp, grp % _RING)

        pltpu.sync_copy(z64_hbm, zrow)
        pltpu.sync_copy(z16_hbm, z16v)
        pltpu.sync_copy(o16_hbm, onesv)

        def zbody(jz, carry):
            base = s * rpt + jz * zc
            pltpu.sync_copy(zrow.at[pl.ds(0, zc)], acc.at[pl.ds(base, zc)])
            pltpu.sync_copy(z16v.at[pl.ds(0, zc)], dacc.at[pl.ds(base, zc)])
            return carry
        lax.fori_loop(0, rpt // zc, zbody, 0)
        plsc.subcore_barrier()

        wait_prefetch(0)
        transform(0)

        def step(grp, k):
            rows = rowbufs[k % 2]
            fire_gathers(k, rows)

            @pl.when(grp + 1 < ngroups)
            def _():
                wait_prefetch((k + 1) % _RING)
            transform((k + 1) % _RING)
            drain_and_scatter(k, rows)

            @pl.when(grp >= 1)
            def _():
                drain_scatters((k - 1) % _RING, rowbufs[(k - 1) % 2])

            @pl.when(grp + _RING - 1 < ngroups)
            def _():
                start_prefetch(grp + _RING - 1, (k - 1) % _RING)

        def ring_body(p, carry):
            for k in range(_RING):
                step(p * _RING + k, k)
            return carry
        lax.fori_loop(0, ngroups // _RING, ring_body, 0)
        drain_scatters((ngroups - 1) % _RING, rowbufs[(ngroups - 1) % 2])
        plsc.subcore_barrier()

        base = s * rpt
        pltpu.sync_copy(acc.at[pl.ds(base, rpt)],
                        agg_hbm.at[c, pl.ds(base, rpt)])
        pltpu.sync_copy(dacc.at[pl.ds(base, rpt)],
                        deg_hbm.at[c, pl.ds(base, rpt)])

    return pl.kernel(
        body,
        out_type=[
            jax.ShapeDtypeStruct((_NC, n_dst, _HALF), jnp.float32),
            jax.ShapeDtypeStruct((_NC, n_dst, _DEGW), jnp.float32),
        ],
        mesh=mesh,
        scratch_types=[
            pltpu.VMEM((_RING, g, _CHUNK), jnp.int32),
            pltpu.VMEM((_RING, g, _CHUNK), jnp.int32),
            pltpu.VMEM((g, _CHUNK, _HALF), jnp.float32),
            pltpu.VMEM((g, _CHUNK, _HALF), jnp.float32),
            pltpu.VMEM((_CHUNK, _DEGW), jnp.float32),
            pltpu.VMEM((128, _HALF), jnp.float32),
            pltpu.VMEM((128, _DEGW), jnp.float32),
            pltpu.VMEM_SHARED((n_dst, _HALF), jnp.float32),
            pltpu.VMEM_SHARED((n_dst, _DEGW), jnp.float32),
            pltpu.SemaphoreType.DMA,
            pltpu.SemaphoreType.DMA,
            pltpu.SemaphoreType.DMA,
            pltpu.SemaphoreType.DMA,
        ],
        compiler_params=pltpu.CompilerParams(use_tc_tiling_on_sc=False),
    )


@functools.lru_cache(maxsize=None)
def _dense_call(n_rows, relu, blk):
    def body(xd, a_a, a_b, dg_a, dg_b, ws, wnt, wnb, b, out):
        m = (jnp.dot(a_a[...], wnt[...], preferred_element_type=jnp.float32)
             + jnp.dot(a_b[...], wnb[...], preferred_element_type=jnp.float32))
        deg = jnp.maximum(dg_a[...][:, 0:1] + dg_b[...][:, 0:1], 1.0)
        r = (jnp.dot(xd[...], ws[...], preferred_element_type=jnp.float32)
             + m / deg + b[...])
        out[...] = jnp.maximum(r, 0.0) if relu else r

    return pl.pallas_call(
        body,
        grid=(n_rows // blk,),
        in_specs=[
            pl.BlockSpec((blk, 128), lambda i: (i, 0)),
            pl.BlockSpec((blk, _HALF), lambda i: (i, 0)),
            pl.BlockSpec((blk, _HALF), lambda i: (i, 0)),
            pl.BlockSpec((blk, _DEGW), lambda i: (i, 0)),
            pl.BlockSpec((blk, _DEGW), lambda i: (i, 0)),
            pl.BlockSpec((128, 128), lambda i: (0, 0)),
            pl.BlockSpec((_HALF, 128), lambda i: (0, 0)),
            pl.BlockSpec((_HALF, 128), lambda i: (0, 0)),
            pl.BlockSpec((1, 128), lambda i: (0, 0)),
        ],
        out_specs=pl.BlockSpec((blk, 128), lambda i: (i, 0)),
        out_shape=jax.ShapeDtypeStruct((n_rows, 128), jnp.float32),
    )


def kernel(x, src0, dst0, src1, dst1, n_dst0, n_dst1,
           W_self1, W_neigh1, b1, W_self2, W_neigh2, b2):
    del n_dst0, n_dst1
    n1 = src1.shape[0]
    n2 = 1024
    f32 = jnp.float32
    x64 = x.reshape(-1, _HALF)
    z64 = jnp.zeros((128, _HALF), f32)
    z16 = jnp.zeros((128, _DEGW), f32)
    o16 = jnp.ones((_CHUNK, _DEGW), f32)
    src0i = src0.astype(jnp.int32).reshape(-1, _CHUNK)
    dst0i = dst0.astype(jnp.int32).reshape(-1, _CHUNK)
    src1i = src1.astype(jnp.int32).reshape(-1, _CHUNK)
    dst1i = dst1.astype(jnp.int32).reshape(-1, _CHUNK)

    agg1, deg1 = _sc_agg_call(x64.shape[0], src0.shape[0], n1)(
        x64, src0i, dst0i, z64, z16, o16)
    h1 = _dense_call(n1, True, 2048)(
        x[:n1], agg1[0], agg1[1], deg1[0], deg1[1],
        W_self1, W_neigh1[:_HALF], W_neigh1[_HALF:], b1.reshape(1, 128))
    agg2, deg2 = _sc_agg_call(2 * n1, src1.shape[0], n2)(
        h1.reshape(-1, _HALF), src1i, dst1i, z64, z16, o16)
    out = _dense_call(n2, False, 1024)(
        h1[:n2], agg2[0], agg2[1], deg2[0], deg2[1],
        W_self2, W_neigh2[:_HALF], W_neigh2[_HALF:], b2.reshape(1, 128))
    return out
